# baseline (speedup 1.0000x reference)
import functools

import jax
import jax.numpy as jnp
from jax import lax
from jax.experimental import pallas as pl
from jax.experimental.pallas import tpu as pltpu
from jax.experimental.pallas import tpu_sc as plsc

NUM_MED = 2000
NUM_MED_PAD = 2048
LANES = 16
NCHUNK = NUM_MED // LANES
NC, NS = 2, 16
COLS_PER_SUB = 128
N_DIAGS = 128
N_PROCS = 64


def _sc_gather_max(diag_tbl, proc_tbl, diags, procs):
    mesh = plsc.VectorSubcoreMesh(
        core_axis_name="c", subcore_axis_name="s", num_cores=NC, num_subcores=NS
    )

    @functools.partial(
        pl.kernel,
        out_type=jax.ShapeDtypeStruct((NC, 2, NUM_MED_PAD), jnp.float32),
        mesh=mesh,
        scratch_types=[
            pltpu.VMEM((LANES,), jnp.int32),
            pltpu.VMEM((LANES, NUM_MED), jnp.float32),
            pltpu.VMEM((1, NUM_MED_PAD), jnp.float32),
            pltpu.VMEM((6, 1, COLS_PER_SUB), jnp.float32),
            pltpu.VMEM((COLS_PER_SUB,), jnp.float32),
            pltpu.VMEM((COLS_PER_SUB,), jnp.float32),
            pltpu.VMEM_SHARED((6, 1, NUM_MED_PAD), jnp.float32),
            pltpu.SemaphoreType.DMA,
        ],
        compiler_params=pltpu.CompilerParams(use_tc_tiling_on_sc=False),
    )
    def k(diag_hbm, proc_hbm, diags_hbm, procs_hbm, part_hbm,
          idx_v, rows_v, partial_v, red_v, outd_v, outp_v, shared, sem):
        c = lax.axis_index("c")
        s = lax.axis_index("s")

        def gather_reduce(tbl_hbm, idx_hbm, base):
            pltpu.sync_copy(idx_hbm.at[pl.ds(base, LANES)], idx_v)
            pltpu.async_copy(tbl_hbm.at[idx_v], rows_v, sem).wait()

            def body(kk, carry):
                off = pl.multiple_of(kk * LANES, LANES)
                acc = rows_v[0, pl.ds(off, LANES)]
                for r in range(1, LANES):
                    acc = jnp.maximum(acc, rows_v[r, pl.ds(off, LANES)])
                partial_v[0, pl.ds(off, LANES)] = acc
                return carry

            lax.fori_loop(0, NCHUNK, body, 0)
            pltpu.sync_copy(partial_v, shared.at[s])

        @pl.when(s < 4)
        def _():
            gather_reduce(diag_hbm, diags_hbm,
                          pl.multiple_of(c * 64 + s * LANES, LANES))

        @pl.when(jnp.logical_and(s >= 4, s < 6))
        def _():
            gather_reduce(proc_hbm, procs_hbm,
                          pl.multiple_of(c * 32 + (s - 4) * LANES, LANES))

        plsc.subcore_barrier()

        off = pl.multiple_of(s * COLS_PER_SUB, COLS_PER_SUB)
        pltpu.sync_copy(shared.at[:, :, pl.ds(off, COLS_PER_SUB)], red_v)
        for kk in range(COLS_PER_SUB // LANES):
            sl = pl.ds(kk * LANES, LANES)
            d = jnp.maximum(
                jnp.maximum(red_v[0, 0, sl], red_v[1, 0, sl]),
                jnp.maximum(red_v[2, 0, sl], red_v[3, 0, sl]))
            p = jnp.maximum(red_v[4, 0, sl], red_v[5, 0, sl])
            outd_v[sl] = d
            outp_v[sl] = p
        pltpu.sync_copy(outd_v, part_hbm.at[c, 0, pl.ds(off, COLS_PER_SUB)])
        pltpu.sync_copy(outp_v, part_hbm.at[c, 1, pl.ds(off, COLS_PER_SUB)])

    return k(diag_tbl, proc_tbl, diags, procs)


def _tc_combine(part4, pre_prob, hl, ll, wm, wp):
    def body(part_ref, pre_ref, hl_ref, ll_ref, wm_ref, wp_ref, out_ref):
        maxd = jnp.maximum(jnp.maximum(part_ref[0:1, :], part_ref[2:3, :]), 0.0)
        maxp = jnp.maximum(jnp.maximum(part_ref[1:2, :], part_ref[3:4, :]), 0.0)
        minus = jnp.logical_and(maxd < ll_ref[0], maxp < ll_ref[1])
        plus = jnp.logical_and(
            jnp.logical_not(minus),
            jnp.logical_or(maxd > hl_ref[0], maxp > hl_ref[1]),
        )
        delta = wp_ref[0] * plus.astype(jnp.float32) \
            - wm_ref[0] * minus.astype(jnp.float32)
        out_ref[...] = pre_ref[...] + delta

    return pl.pallas_call(
        body,
        out_shape=jax.ShapeDtypeStruct((1, NUM_MED), jnp.float32),
        in_specs=[
            pl.BlockSpec(memory_space=pltpu.MemorySpace.VMEM),
            pl.BlockSpec(memory_space=pltpu.MemorySpace.VMEM),
            pl.BlockSpec(memory_space=pltpu.MemorySpace.SMEM),
            pl.BlockSpec(memory_space=pltpu.MemorySpace.SMEM),
            pl.BlockSpec(memory_space=pltpu.MemorySpace.SMEM),
            pl.BlockSpec(memory_space=pltpu.MemorySpace.SMEM),
        ],
        out_specs=pl.BlockSpec(memory_space=pltpu.MemorySpace.VMEM),
    )(part4, pre_prob, hl, ll, wm, wp)


def kernel(pre_prob, diag_med_effect, proc_med_effect, c1_high_limit,
           c1_low_limit, c1_minus_weight, c1_plus_weight, diags, procs):
    part = _sc_gather_max(diag_med_effect, proc_med_effect, diags, procs)
    part4 = part[:, :, :NUM_MED].reshape(NC * 2, NUM_MED)
    wm = jnp.reshape(c1_minus_weight, (1,))
    wp = jnp.reshape(c1_plus_weight, (1,))
    return _tc_combine(part4, pre_prob, c1_high_limit, c1_low_limit, wm, wp)


# speedup vs baseline: 5.2320x; 5.2320x over previous
import jax
import jax.numpy as jnp
from jax import lax
from jax.experimental import pallas as pl
from jax.experimental.pallas import tpu as pltpu

NUM_MED = 2000
N_DIAGS = 128
N_PROCS = 64
NEG = float(jnp.finfo(jnp.float32).min)


def _tree_max(xs):
    while len(xs) > 1:
        nxt = [jnp.maximum(xs[i], xs[i + 1]) for i in range(0, len(xs) - 1, 2)]
        if len(xs) % 2:
            nxt.append(xs[-1])
        xs = nxt
    return xs[0]


def _body(idx_ref, thr_ref, *refs):
    pre_ref = refs[N_DIAGS + N_PROCS]
    out_ref = refs[N_DIAGS + N_PROCS + 1]
    iota = lax.broadcasted_iota(jnp.int32, (8, NUM_MED), 0)

    def masked(j, ref):
        r = idx_ref[j] % 8
        return jnp.where(iota == r, ref[...], NEG)

    maxd8 = _tree_max([masked(j, refs[j]) for j in range(N_DIAGS)])
    maxp8 = _tree_max(
        [masked(N_DIAGS + j, refs[N_DIAGS + j]) for j in range(N_PROCS)]
    )
    maxd = jnp.maximum(jnp.max(maxd8, axis=0, keepdims=True), 0.0)
    maxp = jnp.maximum(jnp.max(maxp8, axis=0, keepdims=True), 0.0)
    hl0, hl1 = thr_ref[0], thr_ref[1]
    ll0, ll1 = thr_ref[2], thr_ref[3]
    wm, wp = thr_ref[4], thr_ref[5]
    minus = jnp.logical_and(maxd < ll0, maxp < ll1)
    plus = jnp.logical_and(
        jnp.logical_not(minus), jnp.logical_or(maxd > hl0, maxp > hl1)
    )
    delta = wp * plus.astype(jnp.float32) - wm * minus.astype(jnp.float32)
    out_ref[...] = pre_ref[...] + delta


def _row_spec(j):
    return pl.BlockSpec((8, NUM_MED), lambda i, idx, thr, j=j: (idx[j] // 8, 0))


def kernel(pre_prob, diag_med_effect, proc_med_effect, c1_high_limit,
           c1_low_limit, c1_minus_weight, c1_plus_weight, diags, procs):
    idx = jnp.concatenate([diags, procs]).astype(jnp.int32)
    thr = jnp.stack([
        c1_high_limit[0], c1_high_limit[1],
        c1_low_limit[0], c1_low_limit[1],
        jnp.asarray(c1_minus_weight, jnp.float32),
        jnp.asarray(c1_plus_weight, jnp.float32),
    ])
    grid_spec = pltpu.PrefetchScalarGridSpec(
        num_scalar_prefetch=2,
        grid=(1,),
        in_specs=[
            *[_row_spec(j) for j in range(N_DIAGS + N_PROCS)],
            pl.BlockSpec((1, NUM_MED), lambda i, idx, thr: (0, 0)),
        ],
        out_specs=pl.BlockSpec((1, NUM_MED), lambda i, idx, thr: (0, 0)),
    )
    return pl.pallas_call(
        _body,
        grid_spec=grid_spec,
        out_shape=jax.ShapeDtypeStruct((1, NUM_MED), jnp.float32),
    )(idx, thr,
      *([diag_med_effect] * N_DIAGS),
      *([proc_med_effect] * N_PROCS),
      pre_prob)
